# baseline (speedup 1.0000x reference)
import functools
import jax, jax.numpy as jnp
from jax import lax
from jax.experimental import pallas as pl
from jax.experimental.pallas import tpu as pltpu
from jax.experimental.pallas import tpu_sc as plsc

@functools.partial(
    pl.kernel,
    mesh=plsc.VectorSubcoreMesh(core_axis_name="c", subcore_axis_name="s"),
    compiler_params=pltpu.CompilerParams(use_tc_tiling_on_sc=True),
    out_type=jax.ShapeDtypeStruct((4096, 200, 64), jnp.float32),
    scratch_types=[
        pltpu.VMEM((200, 128), jnp.int32),
        pltpu.VMEM((200, 128), jnp.int32),
        pltpu.VMEM((128, 128), jnp.float32),
        pltpu.VMEM((128, 64), jnp.float32),
        pltpu.SemaphoreType.DMA,
    ],
)
def _k(xt_hbm, s_hbm, out_hbm, idx_v, hbuf_v, rows_v, obuf_v, gsem):
    wid = lax.axis_index("s") * 2 + lax.axis_index("c")
    b0 = wid * 128
    pltpu.sync_copy(xt_hbm.at[:, pl.ds(b0, 128)], idx_v)
    def t_body(t, carry):
        def prep_j(j, c2):
            v = idx_v[t, pl.ds(j * 16, 16)]
            hbuf_v[t, pl.ds(j * 16, 16)] = (v & 1) * 64
            idx_v[t, pl.ds(j * 16, 16)] = lax.shift_right_logical(v, 1)
            return c2
        lax.fori_loop(0, 8, prep_j, 0)
        pltpu.async_copy(s_hbm.at[idx_v.at[t]], rows_v, gsem).wait()
        def bg_body(bg, c2):
            hv = hbuf_v[t, pl.ds(bg * 16, 16)]
            for k in range(16):
                b = bg * 16 + k
                hk = hv[k]
                for k2 in range(4):
                    obuf_v[b, pl.ds(k2 * 16, 16)] = rows_v[b, pl.ds(hk + k2 * 16, 16)]
            return c2
        lax.fori_loop(0, 8, bg_body, 0)
        pltpu.sync_copy(obuf_v, out_hbm.at[pl.ds(b0, 128), t])
        return carry
    lax.fori_loop(0, 200, t_body, 0)

def kernel(x, table):
    s = table.reshape(500000, 128)
    xt = x.T.astype(jnp.int32)
    return _k(xt, s)


# speedup vs baseline: 1.8558x; 1.8558x over previous
import functools
import jax, jax.numpy as jnp
from jax import lax
from jax.experimental import pallas as pl
from jax.experimental.pallas import tpu as pltpu
from jax.experimental.pallas import tpu_sc as plsc

@functools.partial(
    pl.kernel,
    mesh=plsc.VectorSubcoreMesh(core_axis_name="c", subcore_axis_name="s"),
    compiler_params=pltpu.CompilerParams(use_tc_tiling_on_sc=True),
    out_type=jax.ShapeDtypeStruct((4096, 200, 64), jnp.float32),
    scratch_types=[
        pltpu.VMEM((200, 128), jnp.int32),
        pltpu.VMEM((128, 64), jnp.float32),
        pltpu.VMEM((128, 64), jnp.float32),
        pltpu.SemaphoreType.DMA,
        pltpu.SemaphoreType.DMA,
        pltpu.SemaphoreType.DMA,
        pltpu.SemaphoreType.DMA,
    ],
)
def _k(xt_hbm, table_hbm, out_hbm, idx_v, rows0, rows1, g0, g1, s0, s1):
    rows = (rows0, rows1)
    gsem = (g0, g1)
    ssem = (s0, s1)
    wid = lax.axis_index("s") * 2 + lax.axis_index("c")
    b0 = wid * 128
    pltpu.sync_copy(xt_hbm.at[:, pl.ds(b0, 128)], idx_v)

    def start_gathers(t, slot):
        def bg_body(bg, c2):
            vec = idx_v[t, pl.ds(bg * 16, 16)]
            for k in range(16):
                vk = vec[k]
                pltpu.async_copy(
                    table_hbm.at[vk], rows[slot].at[bg * 16 + k], gsem[slot]
                )
            return c2
        lax.fori_loop(0, 8, bg_body, 0)

    def wait_gathers(t, slot):
        def w_body(j, c2):
            pltpu.make_async_copy(
                table_hbm.at[0], rows[slot].at[0], gsem[slot]
            ).wait()
            return c2
        lax.fori_loop(0, 128, w_body, 0)

    def start_store(t, slot):
        pltpu.async_copy(rows[slot], out_hbm.at[pl.ds(b0, 128), t], ssem[slot])

    def wait_store(t, slot):
        pltpu.make_async_copy(
            rows[slot], out_hbm.at[pl.ds(b0, 128), t], ssem[slot]
        ).wait()

    start_gathers(0, 0)
    start_gathers(1, 1)

    def pair_body(p, carry):
        for slot in (0, 1):
            t = p * 2 + slot
            wait_gathers(t, slot)

            @pl.when(p > 0)
            def _(slot=slot, t=t):
                wait_store(t - 2, slot)

            start_store(t, slot)

            @pl.when(t + 2 < 200)
            def _(slot=slot, t=t):
                start_gathers(t + 2, slot)
        return carry

    lax.fori_loop(0, 100, pair_body, 0)
    wait_store(198, 0)
    wait_store(199, 1)

def kernel(x, table):
    xt = x.T.astype(jnp.int32)
    return _k(xt, table)
